# baseline (speedup 1.0000x reference)
import jax
import jax.numpy as jnp
from jax import lax
from jax.experimental import pallas as pl
from jax.experimental.pallas import tpu as pltpu
from jax.experimental.pallas import tpu_sc as plsc

_D = 64
_B = 16384
_V = 1000000

_info = plsc.get_sparse_core_info()
_NC, _NS = _info.num_cores, _info.num_subcores
_NW = _NC * _NS
_BPW = _B // _NW
_C = 128
_NCH = _BPW // _C


def _gather_body(idx_hbm, tview_hbm, out_hbm, idx_v, q_v, rows2_v, sel_v, sem):
    wid = lax.axis_index("s") * _NC + lax.axis_index("c")
    base = wid * _BPW
    pltpu.sync_copy(idx_hbm.at[pl.ds(base, _BPW)], idx_v)

    def split(g, carry):
        vec = idx_v[pl.ds(g * 16, 16)]
        q_v[pl.ds(g * 16, 16)] = vec >> 1
        return carry

    lax.fori_loop(0, _BPW // 16, split, 0)

    pltpu.async_copy(tview_hbm.at[q_v], rows2_v, sem).wait()

    def chunk(c, carry):
        def select(g, carry2):
            pvec = idx_v[pl.ds(c * _C + g * 16, 16)] & 1
            for k in range(16):
                j = g * 16 + k
                off = pvec[k] * _D
                for q in range(_D // 16):
                    sel_v[j, pl.ds(q * 16, 16)] = rows2_v[
                        c * _C + j, pl.ds(off + q * 16, 16)
                    ]
            return carry2

        lax.fori_loop(0, _C // 16, select, 0)
        pltpu.sync_copy(sel_v, out_hbm.at[pl.ds(base + c * _C, _C)])
        return carry

    lax.fori_loop(0, _NCH, chunk, 0)


@jax.jit
def kernel(elements, entity_embeddings):
    idx = elements.astype(jnp.int32)
    tview = entity_embeddings.reshape(_V // 2, 2 * _D)
    mesh = plsc.VectorSubcoreMesh(core_axis_name="c", subcore_axis_name="s")
    f = pl.kernel(
        _gather_body,
        mesh=mesh,
        out_type=jax.ShapeDtypeStruct((_B, _D), jnp.float32),
        scratch_types=[
            pltpu.VMEM((_BPW,), jnp.int32),
            pltpu.VMEM((_BPW,), jnp.int32),
            pltpu.VMEM((_BPW, 2 * _D), jnp.float32),
            pltpu.VMEM((_C, _D), jnp.float32),
            pltpu.SemaphoreType.DMA,
        ],
    )
    return f(idx, tview)
